# baseline (speedup 1.0000x reference)
import functools

import numpy as np
import jax
import jax.numpy as jnp
from jax import lax
from jax.experimental import pallas as pl
from jax.experimental.pallas import tpu as pltpu

_T, _B = 16, 1024
_ROWS = _T * _B
_D = 1024
_E = 512
_K = 512
_NA = 10
_R = 128
_GB = _B // _R

_DN = (((1,), (0,)), ((), ()))
_BF = jnp.bfloat16
_F32 = jnp.float32


def _dot1(a, b):
    return lax.dot_general(a.astype(_BF), b, dimension_numbers=_DN,
                           preferred_element_type=_F32)


def _dot2(a, b):
    hi = a.astype(_BF)
    lo = (a - hi.astype(_F32)).astype(_BF)
    return (lax.dot_general(hi, b, dimension_numbers=_DN,
                            preferred_element_type=_F32)
            + lax.dot_general(lo, b, dimension_numbers=_DN,
                              preferred_element_type=_F32))


@functools.lru_cache(maxsize=1)
def _gumbel_const():
    return jax.block_until_ready(
        jax.random.gumbel(jax.random.key(12345), (_T, _B, _NA), _F32))


def _body(g_ref, s_ref, mv_ref, sw_ref, mk_ref,
          wmq_ref, wsq_ref, wmkt_ref, wskt_ref,
          wmk_ref, wsk_ref, wp1_ref, bp1_ref, wp2_ref, bp2_ref,
          idx_ref, lg_ref, pol_ref, out_ref):
    s = s_ref[0]
    mq = _dot1(s, wmq_ref[...])
    sq = _dot1(s, wsq_ref[...])
    mqk = _dot2(mq, wmkt_ref[...])
    sqk = _dot2(sq, wskt_ref[...])

    mvb = mv_ref[0].astype(_BF).astype(_F32)
    swb = sw_ref[0].astype(_BF).astype(_F32)
    ml = jnp.sum(mvb * mqk[:, None, :], axis=2)
    sl = jnp.sum(swb * sqk[:, None, :], axis=2)
    raw = jnp.concatenate([ml, sl], axis=1)
    logits = raw / np.sqrt(_K)
    logits = logits / (_NA ** 0.5)
    lg_ref[0] = logits

    maskf = mk_ref[0]
    neg = jnp.finfo(_F32).min
    masked = jnp.where(maskf > 0, logits, neg)
    m = jnp.max(masked, axis=1, keepdims=True)
    ex = jnp.exp(masked - m) * maskf
    ssum = jnp.sum(ex, axis=1, keepdims=True)
    policy = ex / ssum
    pol_ref[0] = policy

    z = jnp.log(policy + 1e-30) + g_ref[0]
    zmax = jnp.max(z, axis=1, keepdims=True)
    iot = lax.broadcasted_iota(jnp.int32, (_R, _NA), 1)
    idx = jnp.min(jnp.where(z == zmax, iot, _NA), axis=1, keepdims=True)
    idx_ref[0] = idx

    idx3 = idx[:, :, None]
    im = lax.broadcasted_iota(jnp.int32, (1, 4, 1), 1)
    e_m = jnp.sum(jnp.where(idx3 == im, mvb, 0.0), axis=1)
    isw = lax.broadcasted_iota(jnp.int32, (1, 6, 1), 1) + 4
    e_s = jnp.sum(jnp.where(idx3 == isw, swb, 0.0), axis=1)

    key = _dot1(e_m, wmk_ref[...]) + _dot1(e_s, wsk_ref[...])
    h = jnp.maximum(_dot1(key, wp1_ref[...]) + bp1_ref[...], 0.0)
    out_ref[0] = s + (_dot1(h, wp2_ref[...]) + bp2_ref[...])


def _full_spec(r, c):
    return pl.BlockSpec((r, c), lambda t, j: (0, 0))


@jax.jit
def _run(g, s, mv, sw, mk, wmq, wsq, wmkt, wskt, wmk, wsk, wp1, bp1, wp2, bp2):
    grid = (_T, _GB)
    return pl.pallas_call(
        _body,
        grid=grid,
        in_specs=[
            pl.BlockSpec((1, _R, _NA), lambda t, j: (t, j, 0)),
            pl.BlockSpec((1, _R, _D), lambda t, j: (t, j, 0)),
            pl.BlockSpec((1, _R, 4, _E), lambda t, j: (t, j, 0, 0)),
            pl.BlockSpec((1, _R, 6, _E), lambda t, j: (t, j, 0, 0)),
            pl.BlockSpec((1, _R, _NA), lambda t, j: (t, j, 0)),
            _full_spec(_D, _K),
            _full_spec(_D, _K),
            _full_spec(_K, _E),
            _full_spec(_K, _E),
            _full_spec(_E, _K),
            _full_spec(_E, _K),
            _full_spec(_K, _E),
            _full_spec(1, _E),
            _full_spec(_E, _D),
            _full_spec(1, _D),
        ],
        out_specs=[
            pl.BlockSpec((1, _R, 1), lambda t, j: (t, j, 0)),
            pl.BlockSpec((1, _R, _NA), lambda t, j: (t, j, 0)),
            pl.BlockSpec((1, _R, _NA), lambda t, j: (t, j, 0)),
            pl.BlockSpec((1, _R, _D), lambda t, j: (t, j, 0)),
        ],
        out_shape=[
            jax.ShapeDtypeStruct((_T, _B, 1), jnp.int32),
            jax.ShapeDtypeStruct((_T, _B, _NA), _F32),
            jax.ShapeDtypeStruct((_T, _B, _NA), _F32),
            jax.ShapeDtypeStruct((_T, _B, _D), _F32),
        ],
        compiler_params=pltpu.CompilerParams(
            dimension_semantics=("arbitrary", "arbitrary"),
        ),
    )(g, s, mv, sw, mk, wmq, wsq, wmkt, wskt, wmk, wsk, wp1, bp1, wp2, bp2)


def kernel(state_emb, moves, switches, move_mask, switch_mask,
           W_mq, W_mk, W_sq, W_sk, W_p1, b_p1, W_p2, b_p2):
    mv = jnp.squeeze(moves, axis=2)
    mk = jnp.concatenate([move_mask, switch_mask], axis=2).astype(_F32)
    g = _gumbel_const()
    wmk = W_mk.astype(_BF)
    wsk = W_sk.astype(_BF)
    idx, logits, policy, out = _run(
        g, state_emb, mv, switches, mk,
        W_mq.astype(_BF), W_sq.astype(_BF),
        wmk.T, wsk.T, wmk, wsk,
        W_p1.astype(_BF), b_p1[None, :], W_p2.astype(_BF), b_p2[None, :])
    return (idx, logits, policy, out)


# speedup vs baseline: 1.1662x; 1.1662x over previous
import functools

import numpy as np
import jax
import jax.numpy as jnp
from jax import lax
from jax.experimental import pallas as pl
from jax.experimental.pallas import tpu as pltpu

_T, _B = 16, 1024
_ROWS = _T * _B
_D = 1024
_E = 512
_K = 512
_NA = 10
_R = 512
_GB = _B // _R

_DN = (((1,), (0,)), ((), ()))
_BF = jnp.bfloat16
_F32 = jnp.float32


def _dot1(a, b):
    return lax.dot_general(a.astype(_BF), b, dimension_numbers=_DN,
                           preferred_element_type=_F32)


def _dot2(a, b):
    hi = a.astype(_BF)
    lo = (a - hi.astype(_F32)).astype(_BF)
    return (lax.dot_general(hi, b, dimension_numbers=_DN,
                            preferred_element_type=_F32)
            + lax.dot_general(lo, b, dimension_numbers=_DN,
                              preferred_element_type=_F32))


@functools.lru_cache(maxsize=1)
def _gumbel_const():
    return jax.block_until_ready(
        jax.random.gumbel(jax.random.key(12345), (_T, _B, _NA), _F32))


def _body(g_ref, s_ref, mv_ref, sw_ref, mk_ref,
          wmq_ref, wsq_ref, wmkt_ref, wskt_ref,
          wmk_ref, wsk_ref, wp1_ref, bp1_ref, wp2_ref, bp2_ref,
          idx_ref, lg_ref, pol_ref, out_ref):
    s = s_ref[0]
    mq = _dot1(s, wmq_ref[...])
    sq = _dot1(s, wsq_ref[...])
    mqk = _dot2(mq, wmkt_ref[...])
    sqk = _dot2(sq, wskt_ref[...])

    mvb = mv_ref[0].astype(_BF).astype(_F32)
    swb = sw_ref[0].astype(_BF).astype(_F32)
    ml = jnp.sum(mvb * mqk[:, None, :], axis=2)
    sl = jnp.sum(swb * sqk[:, None, :], axis=2)
    raw = jnp.concatenate([ml, sl], axis=1)
    logits = raw / np.sqrt(_K)
    logits = logits / (_NA ** 0.5)
    lg_ref[0] = logits

    maskf = mk_ref[0]
    neg = jnp.finfo(_F32).min
    masked = jnp.where(maskf > 0, logits, neg)
    m = jnp.max(masked, axis=1, keepdims=True)
    ex = jnp.exp(masked - m) * maskf
    ssum = jnp.sum(ex, axis=1, keepdims=True)
    policy = ex / ssum
    pol_ref[0] = policy

    z = jnp.log(policy + 1e-30) + g_ref[0]
    zmax = jnp.max(z, axis=1, keepdims=True)
    iot = lax.broadcasted_iota(jnp.int32, (_R, _NA), 1)
    idx = jnp.min(jnp.where(z == zmax, iot, _NA), axis=1, keepdims=True)
    idx_ref[0] = idx

    idx3 = idx[:, :, None]
    im = lax.broadcasted_iota(jnp.int32, (1, 4, 1), 1)
    e_m = jnp.sum(jnp.where(idx3 == im, mvb, 0.0), axis=1)
    isw = lax.broadcasted_iota(jnp.int32, (1, 6, 1), 1) + 4
    e_s = jnp.sum(jnp.where(idx3 == isw, swb, 0.0), axis=1)

    key = _dot1(e_m, wmk_ref[...]) + _dot1(e_s, wsk_ref[...])
    h = jnp.maximum(_dot1(key, wp1_ref[...]) + bp1_ref[...], 0.0)
    out_ref[0] = s + (_dot1(h, wp2_ref[...]) + bp2_ref[...])


def _full_spec(r, c):
    return pl.BlockSpec((r, c), lambda t, j: (0, 0))


@jax.jit
def _run(g, s, mv, sw, mk, wmq, wsq, wmkt, wskt, wmk, wsk, wp1, bp1, wp2, bp2):
    grid = (_T, _GB)
    return pl.pallas_call(
        _body,
        grid=grid,
        in_specs=[
            pl.BlockSpec((1, _R, _NA), lambda t, j: (t, j, 0)),
            pl.BlockSpec((1, _R, _D), lambda t, j: (t, j, 0)),
            pl.BlockSpec((1, _R, 4, _E), lambda t, j: (t, j, 0, 0)),
            pl.BlockSpec((1, _R, 6, _E), lambda t, j: (t, j, 0, 0)),
            pl.BlockSpec((1, _R, _NA), lambda t, j: (t, j, 0)),
            _full_spec(_D, _K),
            _full_spec(_D, _K),
            _full_spec(_K, _E),
            _full_spec(_K, _E),
            _full_spec(_E, _K),
            _full_spec(_E, _K),
            _full_spec(_K, _E),
            _full_spec(1, _E),
            _full_spec(_E, _D),
            _full_spec(1, _D),
        ],
        out_specs=[
            pl.BlockSpec((1, _R, 1), lambda t, j: (t, j, 0)),
            pl.BlockSpec((1, _R, _NA), lambda t, j: (t, j, 0)),
            pl.BlockSpec((1, _R, _NA), lambda t, j: (t, j, 0)),
            pl.BlockSpec((1, _R, _D), lambda t, j: (t, j, 0)),
        ],
        out_shape=[
            jax.ShapeDtypeStruct((_T, _B, 1), jnp.int32),
            jax.ShapeDtypeStruct((_T, _B, _NA), _F32),
            jax.ShapeDtypeStruct((_T, _B, _NA), _F32),
            jax.ShapeDtypeStruct((_T, _B, _D), _F32),
        ],
        compiler_params=pltpu.CompilerParams(
            dimension_semantics=("arbitrary", "arbitrary"),
        ),
    )(g, s, mv, sw, mk, wmq, wsq, wmkt, wskt, wmk, wsk, wp1, bp1, wp2, bp2)


def kernel(state_emb, moves, switches, move_mask, switch_mask,
           W_mq, W_mk, W_sq, W_sk, W_p1, b_p1, W_p2, b_p2):
    mv = jnp.squeeze(moves, axis=2)
    mk = jnp.concatenate([move_mask, switch_mask], axis=2).astype(_F32)
    g = _gumbel_const()
    wmk = W_mk.astype(_BF)
    wsk = W_sk.astype(_BF)
    idx, logits, policy, out = _run(
        g, state_emb, mv, switches, mk,
        W_mq.astype(_BF), W_sq.astype(_BF),
        wmk.T, wsk.T, wmk, wsk,
        W_p1.astype(_BF), b_p1[None, :], W_p2.astype(_BF), b_p2[None, :])
    return (idx, logits, policy, out)


# speedup vs baseline: 1.3729x; 1.1772x over previous
import functools

import numpy as np
import jax
import jax.numpy as jnp
from jax import lax
from jax.experimental import pallas as pl
from jax.experimental.pallas import tpu as pltpu

_T, _B = 16, 1024
_ROWS = _T * _B
_D = 1024
_E = 512
_K = 512
_NA = 10
_R = 256

_DN = (((1,), (0,)), ((), ()))
_BF = jnp.bfloat16
_F32 = jnp.float32


def _dot1(a, b):
    return lax.dot_general(a.astype(_BF), b, dimension_numbers=_DN,
                           preferred_element_type=_F32)


@functools.lru_cache(maxsize=1)
def _gumbel_const():
    return jax.block_until_ready(
        jax.random.gumbel(jax.random.key(12345), (_ROWS, _NA), _F32))


def _body(g_ref, s_ref, mv_ref, sw_ref, mk_ref,
          wmq_ref, wsq_ref, wmk_ref, wsk_ref,
          wp1_ref, bp1_ref, wp2_ref, bp2_ref,
          idx_ref, lg_ref, pol_ref, out_ref):
    s = s_ref[...]
    mq = _dot1(s, wmq_ref[...])
    sq = _dot1(s, wsq_ref[...])

    keys = []
    for n in range(4):
        keys.append(_dot1(mv_ref[:, n * _E:(n + 1) * _E], wmk_ref[...]))
    for n in range(6):
        keys.append(_dot1(sw_ref[:, n * _E:(n + 1) * _E], wsk_ref[...]))

    cols = []
    for n in range(4):
        cols.append(jnp.sum(mq * keys[n], axis=1, keepdims=True))
    for n in range(4, 10):
        cols.append(jnp.sum(sq * keys[n], axis=1, keepdims=True))
    raw = jnp.concatenate(cols, axis=1)
    logits = raw / np.sqrt(_K)
    logits = logits / (_NA ** 0.5)
    lg_ref[...] = logits

    maskf = mk_ref[...]
    neg = jnp.finfo(_F32).min
    masked = jnp.where(maskf > 0, logits, neg)
    m = jnp.max(masked, axis=1, keepdims=True)
    ex = jnp.exp(masked - m) * maskf
    ssum = jnp.sum(ex, axis=1, keepdims=True)
    policy = ex / ssum
    pol_ref[...] = policy

    z = jnp.log(policy + 1e-30) + g_ref[...]
    zmax = jnp.max(z, axis=1, keepdims=True)
    iot = lax.broadcasted_iota(jnp.int32, (_R, _NA), 1)
    idx = jnp.min(jnp.where(z == zmax, iot, _NA), axis=1, keepdims=True)
    idx_ref[...] = idx

    zero = jnp.float32(0.0)
    sel = jnp.zeros((_R, _K), _F32)
    for n in range(10):
        sel = sel + jnp.where(idx == n, keys[n], zero)

    h = jnp.maximum(_dot1(sel, wp1_ref[...]) + bp1_ref[...], 0.0)
    out_ref[...] = s + (_dot1(h, wp2_ref[...]) + bp2_ref[...])


def _row_spec(cols):
    return pl.BlockSpec((_R, cols), lambda i: (i, 0))


def _full_spec(r, c):
    return pl.BlockSpec((r, c), lambda i: (0, 0))


@jax.jit
def _run(g, s, mv, sw, mk, wmq, wsq, wmk, wsk, wp1, bp1, wp2, bp2):
    grid = (_ROWS // _R,)
    return pl.pallas_call(
        _body,
        grid=grid,
        in_specs=[
            _row_spec(_NA),
            _row_spec(_D),
            _row_spec(4 * _E),
            _row_spec(6 * _E),
            _row_spec(_NA),
            _full_spec(_D, _K),
            _full_spec(_D, _K),
            _full_spec(_E, _K),
            _full_spec(_E, _K),
            _full_spec(_K, _E),
            _full_spec(1, _E),
            _full_spec(_E, _D),
            _full_spec(1, _D),
        ],
        out_specs=[
            _row_spec(1),
            _row_spec(_NA),
            _row_spec(_NA),
            _row_spec(_D),
        ],
        out_shape=[
            jax.ShapeDtypeStruct((_ROWS, 1), jnp.int32),
            jax.ShapeDtypeStruct((_ROWS, _NA), _F32),
            jax.ShapeDtypeStruct((_ROWS, _NA), _F32),
            jax.ShapeDtypeStruct((_ROWS, _D), _F32),
        ],
        compiler_params=pltpu.CompilerParams(
            dimension_semantics=("arbitrary",),
        ),
    )(g, s, mv, sw, mk, wmq, wsq, wmk, wsk, wp1, bp1, wp2, bp2)


def kernel(state_emb, moves, switches, move_mask, switch_mask,
           W_mq, W_mk, W_sq, W_sk, W_p1, b_p1, W_p2, b_p2):
    s = state_emb.reshape(_ROWS, _D)
    mv = moves.reshape(_ROWS, 4 * _E).astype(_BF)
    sw = switches.reshape(_ROWS, 6 * _E).astype(_BF)
    mk = jnp.concatenate(
        [move_mask.reshape(_ROWS, 4), switch_mask.reshape(_ROWS, 6)],
        axis=1).astype(_F32)
    g = _gumbel_const()
    idx, logits, policy, out = _run(
        g, s, mv, sw, mk,
        W_mq.astype(_BF), W_sq.astype(_BF), W_mk.astype(_BF), W_sk.astype(_BF),
        W_p1.astype(_BF), b_p1[None, :], W_p2.astype(_BF), b_p2[None, :])
    return (idx.reshape(_T, _B, 1),
            logits.reshape(_T, _B, _NA),
            policy.reshape(_T, _B, _NA),
            out.reshape(_T, _B, _D))


# speedup vs baseline: 1.4555x; 1.0602x over previous
import functools

import numpy as np
import jax
import jax.numpy as jnp
from jax import lax
from jax.experimental import pallas as pl
from jax.experimental.pallas import tpu as pltpu

_T, _B = 16, 1024
_ROWS = _T * _B
_D = 1024
_E = 512
_K = 512
_NA = 10
_R = 512

_DN = (((1,), (0,)), ((), ()))
_BF = jnp.bfloat16
_F32 = jnp.float32


def _dot1(a, b):
    return lax.dot_general(a.astype(_BF), b, dimension_numbers=_DN,
                           preferred_element_type=_F32)


@functools.lru_cache(maxsize=1)
def _gumbel_const():
    return jax.block_until_ready(
        jax.random.gumbel(jax.random.key(12345), (_ROWS, _NA), _F32))


def _body(g_ref, s_ref, mv_ref, sw_ref, mk_ref,
          wmq_ref, wsq_ref, wmk_ref, wsk_ref,
          wp1_ref, bp1_ref, wp2_ref, bp2_ref,
          idx_ref, lg_ref, pol_ref, out_ref):
    s = s_ref[...]
    mq = _dot1(s, wmq_ref[...])
    sq = _dot1(s, wsq_ref[...])

    keys = []
    for n in range(4):
        keys.append(_dot1(mv_ref[:, n * _E:(n + 1) * _E], wmk_ref[...]))
    for n in range(6):
        keys.append(_dot1(sw_ref[:, n * _E:(n + 1) * _E], wsk_ref[...]))

    cols = []
    for n in range(4):
        cols.append(jnp.sum(mq * keys[n], axis=1, keepdims=True))
    for n in range(4, 10):
        cols.append(jnp.sum(sq * keys[n], axis=1, keepdims=True))
    raw = jnp.concatenate(cols, axis=1)
    logits = raw / np.sqrt(_K)
    logits = logits / (_NA ** 0.5)
    lg_ref[...] = logits

    maskf = mk_ref[...]
    neg = jnp.finfo(_F32).min
    masked = jnp.where(maskf > 0, logits, neg)
    m = jnp.max(masked, axis=1, keepdims=True)
    ex = jnp.exp(masked - m) * maskf
    ssum = jnp.sum(ex, axis=1, keepdims=True)
    policy = ex / ssum
    pol_ref[...] = policy

    z = jnp.log(policy + 1e-30) + g_ref[...]
    zmax = jnp.max(z, axis=1, keepdims=True)
    iot = lax.broadcasted_iota(jnp.int32, (_R, _NA), 1)
    idx = jnp.min(jnp.where(z == zmax, iot, _NA), axis=1, keepdims=True)
    idx_ref[...] = idx

    zero = jnp.float32(0.0)
    sel = jnp.zeros((_R, _K), _F32)
    for n in range(10):
        sel = sel + jnp.where(idx == n, keys[n], zero)

    h = jnp.maximum(_dot1(sel, wp1_ref[...]) + bp1_ref[...], 0.0)
    out_ref[...] = s + (_dot1(h, wp2_ref[...]) + bp2_ref[...])


def _row_spec(cols):
    return pl.BlockSpec((_R, cols), lambda i: (i, 0))


def _full_spec(r, c):
    return pl.BlockSpec((r, c), lambda i: (0, 0))


@jax.jit
def _run(g, s, mv, sw, mk, wmq, wsq, wmk, wsk, wp1, bp1, wp2, bp2):
    grid = (_ROWS // _R,)
    return pl.pallas_call(
        _body,
        grid=grid,
        in_specs=[
            _row_spec(_NA),
            _row_spec(_D),
            _row_spec(4 * _E),
            _row_spec(6 * _E),
            _row_spec(_NA),
            _full_spec(_D, _K),
            _full_spec(_D, _K),
            _full_spec(_E, _K),
            _full_spec(_E, _K),
            _full_spec(_K, _E),
            _full_spec(1, _E),
            _full_spec(_E, _D),
            _full_spec(1, _D),
        ],
        out_specs=[
            _row_spec(1),
            _row_spec(_NA),
            _row_spec(_NA),
            _row_spec(_D),
        ],
        out_shape=[
            jax.ShapeDtypeStruct((_ROWS, 1), jnp.int32),
            jax.ShapeDtypeStruct((_ROWS, _NA), _F32),
            jax.ShapeDtypeStruct((_ROWS, _NA), _F32),
            jax.ShapeDtypeStruct((_ROWS, _D), _F32),
        ],
        compiler_params=pltpu.CompilerParams(
            dimension_semantics=("arbitrary",),
        ),
    )(g, s, mv, sw, mk, wmq, wsq, wmk, wsk, wp1, bp1, wp2, bp2)


def kernel(state_emb, moves, switches, move_mask, switch_mask,
           W_mq, W_mk, W_sq, W_sk, W_p1, b_p1, W_p2, b_p2):
    s = state_emb.reshape(_ROWS, _D)
    mv = moves.reshape(_ROWS, 4 * _E).astype(_BF)
    sw = switches.reshape(_ROWS, 6 * _E).astype(_BF)
    mk = jnp.concatenate(
        [move_mask.reshape(_ROWS, 4), switch_mask.reshape(_ROWS, 6)],
        axis=1).astype(_F32)
    g = _gumbel_const()
    idx, logits, policy, out = _run(
        g, s, mv, sw, mk,
        W_mq.astype(_BF), W_sq.astype(_BF), W_mk.astype(_BF), W_sk.astype(_BF),
        W_p1.astype(_BF), b_p1[None, :], W_p2.astype(_BF), b_p2[None, :])
    return (idx.reshape(_T, _B, 1),
            logits.reshape(_T, _B, _NA),
            policy.reshape(_T, _B, _NA),
            out.reshape(_T, _B, _D))
